# baseline (speedup 1.0000x reference)
import jax
import jax.numpy as jnp
from jax.experimental import pallas as pl
from jax.experimental.pallas import tpu as pltpu

N = 1140
H = 600
OUT = 300


def _dot(a, b):
    return jax.lax.dot(a, b, preferred_element_type=jnp.float32)


def _bf(a):
    return a.astype(jnp.bfloat16)


def _gcn_body(x_hbm, A_hbm, W1_hbm, b1_ref, W2_hbm, b2_ref,
              W3a_hbm, W3b_hbm, b3_ref, W4_hbm, b4_ref, W5_hbm, b5_ref,
              xm_ref, out2_ref,
              x_v, A_v, W1_v, W2_v, W3a_v, W3b_v, W4_v, W5_v, sems):
    copies = []
    for i, (src, dst) in enumerate((
            (x_hbm, x_v), (W1_hbm, W1_v), (A_hbm, A_v), (W2_hbm, W2_v),
            (W3a_hbm, W3a_v), (W3b_hbm, W3b_v), (W4_hbm, W4_v),
            (W5_hbm, W5_v))):
        cp = pltpu.make_async_copy(src, dst, sems.at[i])
        cp.start()
        copies.append(cp)
    c_x, c_W1, c_A, c_W2, c_W3a, c_W3b, c_W4, c_W5 = copies

    c_x.wait()
    c_W1.wait()
    t1 = _bf(_dot(_bf(x_v[...]), _bf(W1_v[...])))
    c_A.wait()
    A = _bf(A_v[...])
    x1 = jnp.maximum(_dot(A, t1) + b1_ref[...], 0.0)
    x1b = _bf(x1)
    c_W2.wait()
    x2 = _dot(A, _bf(_dot(x1b, _bf(W2_v[...])))) + b2_ref[...]
    c_W3a.wait()
    c_W3b.wait()
    xm = _dot(_bf(x2), _bf(W3a_v[...])) + _dot(x1b, _bf(W3b_v[...])) + b3_ref[...]
    xm_ref[...] = xm
    c_W4.wait()
    h = _bf(_dot(_bf(_dot(A, _bf(xm))), _bf(W4_v[...])) + b4_ref[...])
    c_W5.wait()
    out2_ref[...] = jax.nn.sigmoid(_dot(_bf(_dot(A, h)), _bf(W5_v[...])) + b5_ref[...])


def kernel(x, A, W1, b1, W2, b2, W3, b3, W4, b4, W5, b5):
    args = (
        x, A,
        W1, b1.reshape(1, H),
        W2, b2.reshape(1, OUT),
        W3[:, :OUT].T, W3[:, OUT:].T, b3.reshape(1, OUT),
        W4, b4.reshape(1, H),
        W5, b5.reshape(1, N),
    )
    hbm = pl.BlockSpec(memory_space=pl.ANY)
    vmem = pl.BlockSpec(memory_space=pltpu.MemorySpace.VMEM)
    return pl.pallas_call(
        _gcn_body,
        in_specs=[hbm, hbm, hbm, vmem, hbm, vmem, hbm, hbm, vmem, hbm, vmem,
                  hbm, vmem],
        out_shape=(
            jax.ShapeDtypeStruct((N, OUT), jnp.float32),
            jax.ShapeDtypeStruct((N, N), jnp.float32),
        ),
        scratch_shapes=[
            pltpu.VMEM((N, N), jnp.float32),
            pltpu.VMEM((N, N), jnp.float32),
            pltpu.VMEM((N, H), jnp.float32),
            pltpu.VMEM((H, OUT), jnp.float32),
            pltpu.VMEM((OUT, OUT), jnp.float32),
            pltpu.VMEM((H, OUT), jnp.float32),
            pltpu.VMEM((OUT, H), jnp.float32),
            pltpu.VMEM((H, N), jnp.float32),
            pltpu.SemaphoreType.DMA((8,)),
        ],
    )(*args)
